# baseline (speedup 1.0000x reference)
import functools

import jax
import jax.numpy as jnp
from jax.experimental import pallas as pl
from jax.experimental.pallas import tpu as pltpu

_BI = 80


def _fused_kernel(seq_ref, w_ref, adj_ref, gamma_ref, beta_ref,
                  pw_ref, out_ref, x_ref, sum_ref, sq_ref):
    i = pl.program_id(0)
    ni = pl.num_programs(0)

    @pl.when(i == 0)
    def _compute_x():
        x_ref[...] = jax.lax.dot_general(
            seq_ref[...], w_ref[...],
            dimension_numbers=(((1,), (1,)), ((), ())),
            preferred_element_type=jnp.float32,
        )

    y_blk = jnp.dot(adj_ref[...], x_ref[...],
                    preferred_element_type=jnp.float32)
    out_ref[pl.ds(i * _BI, _BI), :] = y_blk

    s = jnp.sum(y_blk, axis=0, keepdims=True)
    q = jnp.sum(y_blk * y_blk, axis=0, keepdims=True)

    @pl.when(i == 0)
    def _init_stats():
        sum_ref[...] = s
        sq_ref[...] = q

    @pl.when(i > 0)
    def _acc_stats():
        sum_ref[...] += s
        sq_ref[...] += q

    @pl.when(i == ni - 1)
    def _epilogue():
        n = out_ref.shape[0]
        mean = sum_ref[...] / n
        var = sq_ref[...] / n - mean * mean
        scale = gamma_ref[...] / jnp.sqrt(var + 1e-5)
        o = (out_ref[...] - mean) * scale + beta_ref[...]
        out_ref[...] = jnp.where(o >= 0, o, pw_ref[...] * o)


@functools.partial(jax.jit, static_argnames=("interpret",))
def kernel(seq, adj, W, bias, gamma, beta, prelu_w, interpret=False):
    n, in_ft = seq.shape
    out_ft = W.shape[0]

    gamma2 = gamma.reshape(1, out_ft)
    beta2 = beta.reshape(1, out_ft)
    pw2 = jnp.broadcast_to(prelu_w.reshape(1, -1), (1, out_ft))
    del bias

    const = lambda i: (0, 0)
    out = pl.pallas_call(
        _fused_kernel,
        grid=(n // _BI,),
        in_specs=[
            pl.BlockSpec((n, in_ft), const),
            pl.BlockSpec((out_ft, in_ft), const),
            pl.BlockSpec((_BI, n), lambda i: (i, 0)),
            pl.BlockSpec((1, out_ft), const),
            pl.BlockSpec((1, out_ft), const),
            pl.BlockSpec((1, out_ft), const),
        ],
        out_specs=pl.BlockSpec((n, out_ft), const),
        out_shape=jax.ShapeDtypeStruct((n, out_ft), jnp.float32),
        scratch_shapes=[
            pltpu.VMEM((n, out_ft), jnp.float32),
            pltpu.VMEM((1, out_ft), jnp.float32),
            pltpu.VMEM((1, out_ft), jnp.float32),
        ],
        interpret=interpret,
    )(seq, W, adj, gamma2, beta2, pw2)
    return out


# speedup vs baseline: 1.3690x; 1.3690x over previous
import functools

import jax
import jax.numpy as jnp
from jax.experimental import pallas as pl
from jax.experimental.pallas import tpu as pltpu

_BI = 200


def _fused_kernel(seq_ref, w_ref, adj_ref, gamma_ref, beta_ref,
                  pw_ref, out_ref, x_ref, sum_ref, sq_ref):
    i = pl.program_id(0)
    ni = pl.num_programs(0)

    @pl.when(i == 0)
    def _compute_x():
        x_ref[...] = jax.lax.dot_general(
            seq_ref[...], w_ref[...],
            dimension_numbers=(((1,), (1,)), ((), ())),
            preferred_element_type=jnp.float32,
        )

    y_blk = jnp.dot(adj_ref[...], x_ref[...],
                    preferred_element_type=jnp.float32)
    out_ref[pl.ds(i * _BI, _BI), :] = y_blk

    s = jnp.sum(y_blk, axis=0, keepdims=True)
    q = jnp.sum(y_blk * y_blk, axis=0, keepdims=True)

    @pl.when(i == 0)
    def _init_stats():
        sum_ref[...] = s
        sq_ref[...] = q

    @pl.when(i > 0)
    def _acc_stats():
        sum_ref[...] += s
        sq_ref[...] += q

    @pl.when(i == ni - 1)
    def _epilogue():
        n = out_ref.shape[0]
        mean = sum_ref[...] / n
        var = sq_ref[...] / n - mean * mean
        scale = gamma_ref[...] / jnp.sqrt(var + 1e-5)
        o = (out_ref[...] - mean) * scale + beta_ref[...]
        out_ref[...] = jnp.where(o >= 0, o, pw_ref[...] * o)


@functools.partial(jax.jit, static_argnames=("interpret",))
def kernel(seq, adj, W, bias, gamma, beta, prelu_w, interpret=False):
    n, in_ft = seq.shape
    out_ft = W.shape[0]

    gamma2 = gamma.reshape(1, out_ft)
    beta2 = beta.reshape(1, out_ft)
    pw2 = jnp.broadcast_to(prelu_w.reshape(1, -1), (1, out_ft))
    del bias

    const = lambda i: (0, 0)
    out = pl.pallas_call(
        _fused_kernel,
        grid=(n // _BI,),
        in_specs=[
            pl.BlockSpec((n, in_ft), const),
            pl.BlockSpec((out_ft, in_ft), const),
            pl.BlockSpec((_BI, n), lambda i: (i, 0)),
            pl.BlockSpec((1, out_ft), const),
            pl.BlockSpec((1, out_ft), const),
            pl.BlockSpec((1, out_ft), const),
        ],
        out_specs=pl.BlockSpec((n, out_ft), const),
        out_shape=jax.ShapeDtypeStruct((n, out_ft), jnp.float32),
        scratch_shapes=[
            pltpu.VMEM((n, out_ft), jnp.float32),
            pltpu.VMEM((1, out_ft), jnp.float32),
            pltpu.VMEM((1, out_ft), jnp.float32),
        ],
        interpret=interpret,
    )(seq, W, adj, gamma2, beta2, pw2)
    return out
